# Initial kernel scaffold; baseline (speedup 1.0000x reference)
import jax
import jax.numpy as jnp
from jax.experimental import pallas as pl


def kernel(x, edge_index, pos, batch, W_in0, b_in0, W_out0, b_out0, W_in1, b_in1, W_out1, b_out1, W_in2, b_in2, W_out2, b_out2, W_lin, b_lin):
    raise NotImplementedError("write your pallas kernel here")


# speedup vs baseline: 4.3656x; 4.3656x over previous
import functools

import jax
import jax.numpy as jnp
from jax import lax
from jax.experimental import pallas as pl
from jax.experimental.pallas import tpu as pltpu
from jax.experimental.pallas import tpu_sc as plsc

N = 100000
E = 1600000
G = 256
HID = 64
F = 16
NCLS = 10

NC = 2
NS = 16
NW = NC * NS
CH = 128
NBK = 8
CPW = 392
OUTER = CPW // NBK
EP = NW * CPW * CH
BE = 4096
NH = N // NC
NT = NH + 16
TRT = NT // NS
CPT = (EP // CH) // NS
OUTER2 = CPT // NBK

_f32 = jnp.float32


def _sc_gather_pair_body(t0, srcm, dstm, s16o, d16o, sidx, didx, srows, drows,
                         sem):
  c = lax.axis_index("c")
  s = lax.axis_index("s")
  w = s * NC + c

  def outer(o, carry):
    cb = w * CPW + o * NBK
    pltpu.sync_copy(srcm.at[pl.ds(cb, NBK)], sidx)
    pltpu.sync_copy(dstm.at[pl.ds(cb, NBK)], didx)
    cps = []
    for j in range(NBK):
      cps.append(pltpu.async_copy(t0.at[sidx.at[j]],
                                  srows.at[pl.ds(j * CH, CH)], sem))
      cps.append(pltpu.async_copy(t0.at[didx.at[j]],
                                  drows.at[pl.ds(j * CH, CH)], sem))
    for cp in cps:
      cp.wait()
    eb = cb * CH
    pltpu.sync_copy(srows, s16o.at[pl.ds(eb, NBK * CH)])
    pltpu.sync_copy(drows, d16o.at[pl.ds(eb, NBK * CH)])
    return carry

  lax.fori_loop(0, OUTER, outer, 0)


def _sc_gather_one_body(tab, srcm, xjo, sidx, srows, sem):
  c = lax.axis_index("c")
  s = lax.axis_index("s")
  w = s * NC + c

  def outer(o, carry):
    cb = w * CPW + o * NBK
    pltpu.sync_copy(srcm.at[pl.ds(cb, NBK)], sidx)
    cps = []
    for j in range(NBK):
      cps.append(pltpu.async_copy(tab.at[sidx.at[j]],
                                  srows.at[pl.ds(j * CH, CH)], sem))
    for cp in cps:
      cp.wait()
    pltpu.sync_copy(srows, xjo.at[pl.ds(cb * CH, NBK * CH)])
    return carry

  lax.fori_loop(0, OUTER, outer, 0)


def _sc_scatter_body(dstm, m16, pout, didx, lidx, rows, zbuf, table, sem):
  c = lax.axis_index("c")
  s = lax.axis_index("s")
  base = c * NH

  def zrow(i, carry):
    zbuf[i, :] = jnp.zeros((16,), _f32)
    return carry

  lax.fori_loop(0, TRT, zrow, 0)
  pltpu.sync_copy(zbuf, table.at[pl.ds(s * TRT, TRT)])
  plsc.subcore_barrier()

  lane = lax.broadcasted_iota(jnp.int32, (16,), 0)

  def outer(o, carry):
    cb = s * CPT + o * NBK
    pltpu.sync_copy(dstm.at[pl.ds(cb, NBK)], didx)
    pltpu.sync_copy(m16.at[pl.ds(cb * CH, NBK * CH)], rows)
    for j in range(NBK):
      for k in range(CH // 16):
        v = didx[j, pl.ds(k * 16, 16)] - base
        ok = (v >= 0) & (v < NH)
        lidx[j, pl.ds(k * 16, 16)] = jnp.where(ok, v, NH + lane)
    for j in range(NBK):
      pltpu.sync_copy(rows.at[pl.ds(j * CH, CH)], table.at[lidx.at[j]],
                      add=True)
    return carry

  lax.fori_loop(0, OUTER2, outer, 0)
  plsc.subcore_barrier()
  pltpu.sync_copy(table.at[pl.ds(s * TRT, TRT)],
                  pout.at[c, pl.ds(s * TRT, TRT)])


@functools.lru_cache(maxsize=None)
def _sc_kernels():
  mesh = plsc.VectorSubcoreMesh(core_axis_name="c", subcore_axis_name="s",
                                num_cores=NC, num_subcores=NS)
  cparams = pltpu.CompilerParams(use_tc_tiling_on_sc=False,
                                 has_side_effects=True)
  gather_pair = pl.kernel(
      _sc_gather_pair_body,
      out_type=[jax.ShapeDtypeStruct((EP, 16), _f32),
                jax.ShapeDtypeStruct((EP, 16), _f32)],
      mesh=mesh,
      compiler_params=cparams,
      scratch_types=[
          pltpu.VMEM((NBK, CH), jnp.int32),
          pltpu.VMEM((NBK, CH), jnp.int32),
          pltpu.VMEM((NBK * CH, 16), _f32),
          pltpu.VMEM((NBK * CH, 16), _f32),
          pltpu.SemaphoreType.DMA,
      ],
  )
  gather_one = pl.kernel(
      _sc_gather_one_body,
      out_type=jax.ShapeDtypeStruct((EP, 16), _f32),
      mesh=mesh,
      compiler_params=cparams,
      scratch_types=[
          pltpu.VMEM((NBK, CH), jnp.int32),
          pltpu.VMEM((NBK * CH, 16), _f32),
          pltpu.SemaphoreType.DMA,
      ],
  )
  scatter16 = pl.kernel(
      _sc_scatter_body,
      out_type=jax.ShapeDtypeStruct((NC, NT, 16), _f32),
      mesh=mesh,
      compiler_params=cparams,
      scratch_types=[
          pltpu.VMEM((NBK, CH), jnp.int32),
          pltpu.VMEM((NBK, CH), jnp.int32),
          pltpu.VMEM((NBK * CH, 16), _f32),
          pltpu.VMEM((TRT, 16), _f32),
          pltpu.VMEM_SHARED((NT, 16), _f32),
          pltpu.SemaphoreType.DMA,
      ],
  )
  return gather_pair, gather_one, scatter16


def _tc_dense0_body(s16, d16, wi, bi, wo, out):
  pid = pl.program_id(0)
  sb = s16[...]
  db = d16[...]
  relx = sb[:, 0:1] - db[:, 0:1]
  rely = sb[:, 1:2] - db[:, 1:2]
  e0 = sb[:, 2:3]
  spatial = jnp.maximum(relx * wi[0:1, :] + rely * wi[1:2, :] + bi[...], 0.0)
  m = jnp.dot(spatial, wo[...], preferred_element_type=_f32) * e0
  gi = pid * BE + lax.broadcasted_iota(jnp.int32, (BE, 1), 0)
  out[...] = jnp.where(gi < E, m, 0.0)


def _tc_dense12_body(s16, d16, xj, wi, bi, wo, out):
  pid = pl.program_id(0)
  sb = s16[...]
  db = d16[...]
  relx = sb[:, 0:1] - db[:, 0:1]
  rely = sb[:, 1:2] - db[:, 1:2]
  spatial = jnp.maximum(relx * wi[0:1, :] + rely * wi[1:2, :] + bi[...], 0.0)
  xb = xj[...]
  msg = spatial * jnp.concatenate([xb, xb, xb, xb], axis=1)
  m = jnp.dot(msg, wo[...], preferred_element_type=_f32)
  gi = pid * BE + lax.broadcasted_iota(jnp.int32, (BE, 1), 0)
  out[...] = jnp.where(gi < E, m, 0.0)


BN = 2000


def _tc_combine_body(p, bo, h):
  h[...] = p[0] + bo[...]


BP = 2000
NSTEP = N // BP


def _tc_pool_body(p, bo, seg, wl, bl, out, acc):
  pid = pl.program_id(0)

  @pl.when(pid == 0)
  def _():
    acc[...] = jnp.zeros_like(acc)

  h = p[0] + bo[...]
  oh = (lax.broadcasted_iota(jnp.int32, (G, BP), 0) == seg[0]).astype(_f32)
  acc[...] += jnp.dot(oh, h, preferred_element_type=_f32)

  @pl.when(pid == NSTEP - 1)
  def _():
    logits = jnp.dot(acc[...], wl[...], preferred_element_type=_f32) + bl[...]
    mx = jnp.max(logits, axis=1, keepdims=True)
    lse = jnp.log(jnp.sum(jnp.exp(logits - mx), axis=1, keepdims=True)) + mx
    out[...] = logits - lse


def _dense0_call(s16, d16, wi, bi, wo):
  return pl.pallas_call(
      _tc_dense0_body,
      grid=(EP // BE,),
      in_specs=[
          pl.BlockSpec((BE, 16), lambda i: (i, 0)),
          pl.BlockSpec((BE, 16), lambda i: (i, 0)),
          pl.BlockSpec((2, HID), lambda i: (0, 0)),
          pl.BlockSpec((1, HID), lambda i: (0, 0)),
          pl.BlockSpec((HID, F), lambda i: (0, 0)),
      ],
      out_specs=pl.BlockSpec((BE, 16), lambda i: (i, 0)),
      out_shape=jax.ShapeDtypeStruct((EP, 16), _f32),
      compiler_params=pltpu.CompilerParams(
          dimension_semantics=("parallel",)),
  )(s16, d16, wi, bi, wo)


def _dense12_call(s16, d16, xj, wi, bi, wo):
  return pl.pallas_call(
      _tc_dense12_body,
      grid=(EP // BE,),
      in_specs=[
          pl.BlockSpec((BE, 16), lambda i: (i, 0)),
          pl.BlockSpec((BE, 16), lambda i: (i, 0)),
          pl.BlockSpec((BE, 16), lambda i: (i, 0)),
          pl.BlockSpec((2, HID), lambda i: (0, 0)),
          pl.BlockSpec((1, HID), lambda i: (0, 0)),
          pl.BlockSpec((HID, F), lambda i: (0, 0)),
      ],
      out_specs=pl.BlockSpec((BE, 16), lambda i: (i, 0)),
      out_shape=jax.ShapeDtypeStruct((EP, 16), _f32),
      compiler_params=pltpu.CompilerParams(
          dimension_semantics=("parallel",)),
  )(s16, d16, xj, wi, bi, wo)


def _combine_call(parts, bo):
  return pl.pallas_call(
      _tc_combine_body,
      grid=(N // BN,),
      in_specs=[
          pl.BlockSpec((1, BN, 16), lambda i: (i // (NH // BN), i % (NH // BN), 0)),
          pl.BlockSpec((1, F), lambda i: (0, 0)),
      ],
      out_specs=pl.BlockSpec((BN, 16), lambda i: (i, 0)),
      out_shape=jax.ShapeDtypeStruct((N, 16), _f32),
      compiler_params=pltpu.CompilerParams(
          dimension_semantics=("parallel",)),
  )(parts, bo)


def _pool_call(parts, bo, seg, wl, bl):
  return pl.pallas_call(
      _tc_pool_body,
      grid=(NSTEP,),
      in_specs=[
          pl.BlockSpec((1, BP, 16), lambda i: (i // (NH // BP), i % (NH // BP), 0)),
          pl.BlockSpec((1, F), lambda i: (0, 0)),
          pl.BlockSpec((1, 1, BP), lambda i: (i, 0, 0)),
          pl.BlockSpec((F, NCLS), lambda i: (0, 0)),
          pl.BlockSpec((1, NCLS), lambda i: (0, 0)),
      ],
      out_specs=pl.BlockSpec((G, NCLS), lambda i: (0, 0)),
      out_shape=jax.ShapeDtypeStruct((G, NCLS), _f32),
      scratch_shapes=[pltpu.VMEM((G, F), _f32)],
      compiler_params=pltpu.CompilerParams(
          dimension_semantics=("arbitrary",)),
  )(parts, bo, seg, wl, bl)


def kernel(x, edge_index, pos, batch,
           W_in0, b_in0, W_out0, b_out0,
           W_in1, b_in1, W_out1, b_out1,
           W_in2, b_in2, W_out2, b_out2,
           W_lin, b_lin):
  src = edge_index[0]
  dst = edge_index[1]
  pad = EP - E
  padidx = (jnp.arange(pad, dtype=jnp.int32) * 9973) % N
  srcm = jnp.concatenate([src, padidx]).reshape(EP // CH, CH)
  dstm = jnp.concatenate([dst, padidx]).reshape(EP // CH, CH)
  t0 = jnp.concatenate([pos, x[:, 1:2], jnp.zeros((N, 13), _f32)], axis=1)
  seg = batch.reshape(NSTEP, 1, BP)

  _gather_pair, _gather_one, _scatter16 = _sc_kernels()
  s16, d16 = _gather_pair(t0, srcm, dstm)

  m16 = _dense0_call(s16, d16, W_in0, b_in0.reshape(1, HID), W_out0)
  parts = _scatter16(dstm, m16)
  h = _combine_call(parts, b_out0.reshape(1, F))

  xj = _gather_one(h, srcm)
  m16 = _dense12_call(s16, d16, xj, W_in1, b_in1.reshape(1, HID), W_out1)
  parts = _scatter16(dstm, m16)
  h = _combine_call(parts, b_out1.reshape(1, F))

  xj = _gather_one(h, srcm)
  m16 = _dense12_call(s16, d16, xj, W_in2, b_in2.reshape(1, HID), W_out2)
  parts = _scatter16(dstm, m16)

  return _pool_call(parts, b_out2.reshape(1, F), seg, W_lin,
                    b_lin.reshape(1, NCLS))
